# baseline (speedup 1.0000x reference)
import functools

import jax
import jax.numpy as jnp
import numpy as np
from jax import lax
from jax.experimental import pallas as pl
from jax.experimental.pallas import tpu as pltpu
from jax.experimental.pallas import tpu_sc as plsc

B = 4096
F = 26
V = 100000
D = 64
DENSE = 13
H1 = 1024
H2 = 512

_N = B * F
_P = 2 * D
_NC = 2
_NS = 16
_NW = _NC * _NS
_PER_W = _N // _NW
_CHUNK = 128
_NCHUNK = _PER_W // _CHUNK


def _gather_pair_rows(table2, idx2):
    mesh = plsc.VectorSubcoreMesh(core_axis_name="c", subcore_axis_name="s")

    @functools.partial(
        pl.kernel,
        out_type=jax.ShapeDtypeStruct((_N, _P), jnp.float32),
        mesh=mesh,
        scratch_types=[
            pltpu.VMEM((_PER_W,), jnp.int32),
            pltpu.VMEM((2, _CHUNK, _P), jnp.float32),
            pltpu.SemaphoreType.DMA((2,)),
        ],
    )
    def gather_kernel(tab_hbm, idx_hbm, out_hbm, idx_v, buf, sem):
        wid = lax.axis_index("s") * _NC + lax.axis_index("c")
        base = wid * _PER_W
        pltpu.sync_copy(idx_hbm.at[pl.ds(base, _PER_W)], idx_v)

        pltpu.async_copy(
            tab_hbm.at[idx_v.at[pl.ds(0, _CHUNK)]], buf.at[0], sem.at[0]
        )

        def body(c, carry):
            slot = lax.rem(c, 2)
            nxt = lax.rem(c + 1, 2)

            @pl.when(c + 1 < _NCHUNK)
            def _():
                pltpu.async_copy(
                    tab_hbm.at[idx_v.at[pl.ds((c + 1) * _CHUNK, _CHUNK)]],
                    buf.at[nxt], sem.at[nxt],
                )

            pltpu.make_async_copy(
                tab_hbm.at[pl.ds(0, _CHUNK)], buf.at[slot], sem.at[slot]
            ).wait()
            pltpu.sync_copy(
                buf.at[slot], out_hbm.at[pl.ds(base + c * _CHUNK, _CHUNK)]
            )
            return carry

        lax.fori_loop(0, _NCHUNK, body, 0)

    return gather_kernel(table2, idx2)


_BB = 512


def _mlp_body(e2_ref, par_ref, dense_ref, sel_ref, w1p_ref, w1d_ref, b1_ref,
              w2_ref, b2_ref, w3_ref, b3_ref, out_ref):
    q = jnp.dot(par_ref[...], sel_ref[...], preferred_element_type=jnp.float32)
    hl = (lax.broadcasted_iota(jnp.int32, (1, F * _P), 1) // D) % 2
    keep = q == hl.astype(jnp.float32)
    e2sel = jnp.where(keep, e2_ref[...], 0.0).astype(jnp.bfloat16)
    h = jnp.dot(e2sel, w1p_ref[...], preferred_element_type=jnp.float32)
    h = h + jnp.dot(dense_ref[...], w1d_ref[...],
                    preferred_element_type=jnp.float32)
    h = jnp.maximum(h + b1_ref[...], 0.0).astype(jnp.bfloat16)
    h2 = jnp.dot(h, w2_ref[...], preferred_element_type=jnp.float32)
    h2 = jnp.maximum(h2 + b2_ref[...], 0.0)
    logit = jnp.sum(h2 * w3_ref[...], axis=1, keepdims=True) + b3_ref[...]
    out_ref[...] = jax.nn.sigmoid(logit)


def _mlp(e2, par, dense, sel, W1pair, W1d, b1, W2, b2, w3row, b3):
    return pl.pallas_call(
        _mlp_body,
        grid=(B // _BB,),
        in_specs=[
            pl.BlockSpec((_BB, F * _P), lambda i: (i, 0)),
            pl.BlockSpec((_BB, F), lambda i: (i, 0)),
            pl.BlockSpec((_BB, DENSE), lambda i: (i, 0)),
            pl.BlockSpec((F, F * _P), lambda i: (0, 0)),
            pl.BlockSpec((F * _P, H1), lambda i: (0, 0)),
            pl.BlockSpec((DENSE, H1), lambda i: (0, 0)),
            pl.BlockSpec((1, H1), lambda i: (0, 0)),
            pl.BlockSpec((H1, H2), lambda i: (0, 0)),
            pl.BlockSpec((1, H2), lambda i: (0, 0)),
            pl.BlockSpec((1, H2), lambda i: (0, 0)),
            pl.BlockSpec((1, 1), lambda i: (0, 0)),
        ],
        out_specs=pl.BlockSpec((_BB, 1), lambda i: (i, 0)),
        out_shape=jax.ShapeDtypeStruct((B, 1), jnp.float32),
    )(e2, par, dense, sel, W1pair, W1d, b1, W2, b2, w3row, b3)


_SEL = np.repeat(np.eye(F, dtype=np.float32), _P, axis=1)


def kernel(sparse_ids, dense_feats, tables, W1, b1, W2, b2, W3, b3):
    table2 = tables.reshape(F * V // 2, _P)
    offs = (jnp.arange(F, dtype=jnp.int32) * V)[None, :]
    flat_idx = sparse_ids.astype(jnp.int32) + offs
    idx2 = (flat_idx >> 1).reshape(_N)
    par = (flat_idx & 1).astype(jnp.float32)

    e2 = _gather_pair_rows(table2, idx2).reshape(B, F * _P)

    w1a = W1[:F * D].reshape(F, D, H1)
    W1pair = jnp.concatenate([w1a[:, None], w1a[:, None]], axis=1)
    W1pair = W1pair.reshape(F * _P, H1).astype(jnp.bfloat16)
    W1d = W1[F * D:]
    W2b = W2.astype(jnp.bfloat16)

    return _mlp(e2, par, dense_feats, _SEL, W1pair, W1d,
                b1.reshape(1, H1), W2b, b2.reshape(1, H2),
                W3.reshape(1, H2), b3.reshape(1, 1))
